# baseline (speedup 1.0000x reference)
import functools

import jax
import jax.numpy as jnp
from jax import lax
from jax.experimental import pallas as pl
from jax.experimental.pallas import tpu as pltpu
from jax.experimental.pallas import tpu_sc as plsc

_EMBED = 64
_SCALE = 8.0
_NC, _NS = 2, 16
_NW = _NC * _NS
_IDXW = 128
_CHUNK = 640
_G = _CHUNK // _IDXW


@functools.lru_cache(maxsize=None)
def _make_kernel(n_rows):
    b_per_w = n_rows // _NW
    n_chunks = b_per_w // _CHUNK
    idx_rows = b_per_w // _IDXW

    mesh = plsc.VectorSubcoreMesh(core_axis_name="c", subcore_axis_name="s")

    @functools.partial(
        pl.kernel,
        out_type=jax.ShapeDtypeStruct((n_rows, _EMBED), jnp.float32),
        mesh=mesh,
        scratch_types=[
            pltpu.VMEM((idx_rows, _IDXW), jnp.int32),
            pltpu.VMEM((2, _CHUNK, _EMBED), jnp.float32),
            pltpu.SemaphoreType.DMA,
            pltpu.SemaphoreType.DMA,
            pltpu.SemaphoreType.DMA,
            pltpu.SemaphoreType.DMA,
        ],
        compiler_params=pltpu.CompilerParams(use_tc_tiling_on_sc=False),
    )
    def k(idx_hbm, table_hbm, out_hbm, idx_v, rows_v, sg0, sg1, sw0, sw1):
        cid = lax.axis_index("c")
        sid = lax.axis_index("s")
        wid = sid * _NC + cid
        row0 = wid * b_per_w

        pltpu.sync_copy(idx_hbm.at[pl.ds(wid * idx_rows, idx_rows)], idx_v)

        sg = (sg0, sg1)
        sw = (sw0, sw1)

        def issue_gathers(ci, buf):
            handles = []
            for j in range(_G):
                handles.append(
                    pltpu.async_copy(
                        table_hbm.at[idx_v.at[ci * _G + j]],
                        rows_v.at[buf, pl.ds(j * _IDXW, _IDXW)],
                        sg[buf],
                    )
                )
            return handles

        def scale_and_writeback(ci, buf, gather_handles):
            for h in gather_handles:
                h.wait()

            @pl.loop(0, _CHUNK)
            def _(r):
                for q in range(_EMBED // 16):
                    sl = (buf, r, pl.ds(q * 16, 16))
                    rows_v[sl] = rows_v[sl] * _SCALE

            return pltpu.async_copy(
                rows_v.at[buf],
                out_hbm.at[pl.ds(row0 + ci * _CHUNK, _CHUNK)],
                sw[buf],
            )

        wb = [None, None]
        gh = [None, None]
        gh[0] = issue_gathers(0, 0)
        for ci in range(n_chunks):
            buf = ci % 2
            nbuf = 1 - buf
            if ci + 1 < n_chunks:
                if wb[nbuf] is not None:
                    wb[nbuf].wait()
                gh[nbuf] = issue_gathers(ci + 1, nbuf)
            wb[buf] = scale_and_writeback(ci, buf, gh[buf])
        wb[0].wait()
        wb[1].wait()

    return k


@jax.jit
def kernel(input, table):
    b, s = input.shape
    n = b * s
    idx = input.reshape(n // _IDXW, _IDXW)
    out = _make_kernel(n)(idx, table)
    return out.reshape(b, s, _EMBED)
